# baseline (speedup 1.0000x reference)
import functools

import jax
import jax.numpy as jnp
from jax import lax
from jax.experimental import pallas as pl
from jax.experimental.pallas import tpu as pltpu
from jax.experimental.pallas import tpu_sc as plsc

ROWS = 16384
COLS = 200
D = 64
V = 202
NW = 32
CI = 512
SUB_PER_J = ROWS // CI
N_SUB = COLS * SUB_PER_J
SUB_PER_W = N_SUB // NW
L = 16


def _lookup_kernel(locT, tableT):
    mesh = plsc.VectorSubcoreMesh(core_axis_name="c", subcore_axis_name="s")

    @functools.partial(
        pl.kernel,
        mesh=mesh,
        compiler_params=pltpu.CompilerParams(
            use_tc_tiling_on_sc=False, needs_layout_passes=False),
        out_type=jax.ShapeDtypeStruct((COLS, D, ROWS), jnp.float32),
        scratch_types=[
            pltpu.VMEM((2, CI), jnp.int32),
            pltpu.VMEM((2, D, CI), jnp.float32),
            pltpu.VMEM((D * V,), jnp.float32),
            pltpu.SemaphoreType.DMA,
            pltpu.SemaphoreType.DMA,
            pltpu.SemaphoreType.DMA,
            pltpu.SemaphoreType.DMA,
        ],
    )
    def k(locT_hbm, tblT_hbm, out_hbm, idx_v, outbuf, tbl_v,
          sem_i0, sem_i1, sem_o0, sem_o1):
        wid = lax.axis_index("s") * 2 + lax.axis_index("c")
        sem_i = (sem_i0, sem_i1)
        sem_o = (sem_o0, sem_o1)

        pltpu.sync_copy(tblT_hbm, tbl_v)

        def sub_pos(t):
            s = jnp.minimum(wid * SUB_PER_W + t, N_SUB - 1)
            return s // SUB_PER_J, (s % SUB_PER_J) * CI

        def idx_fetch(t, b):
            j, i0 = sub_pos(t)
            return pltpu.make_async_copy(
                locT_hbm.at[j, pl.ds(i0, CI)], idx_v.at[b], sem_i[b])

        def out_store(t, b):
            j, i0 = sub_pos(t)
            return pltpu.make_async_copy(
                outbuf.at[b], out_hbm.at[j, :, pl.ds(i0, CI)], sem_o[b])

        def compute(b):
            def grp(k16, carry):
                idxreg = idx_v[b, pl.ds(k16 * L, L)]

                @plsc.parallel_loop(0, D, step=1, unroll=8)
                def dloop(d):
                    outbuf[b, d, pl.ds(k16 * L, L)] = plsc.load_gather(
                        tbl_v, [idxreg + d * V])

                return carry
            lax.fori_loop(0, CI // L, grp, 0)

        def step(t, b, first):
            if not first:
                out_store(t, b).wait()
            idx_fetch(t, b).wait()
            compute(b)
            idx_fetch(t + 2, b).start()
            out_store(t, b).start()

        idx_fetch(0, 0).start()
        idx_fetch(1, 1).start()
        step(0, 0, True)
        step(1, 1, True)

        def body(p, carry):
            t = 2 + 2 * p
            step(t, 0, False)
            step(t + 1, 1, False)
            return carry

        lax.fori_loop(0, (SUB_PER_W - 2) // 2, body, 0)

        for b in range(2):
            out_store(0, b).wait()
            idx_fetch(0, b).wait()

    return k(locT, tableT.reshape(-1))


def kernel(location, table):
    locT = location.transpose().astype(jnp.int32)
    tableT = table.transpose()
    outT = _lookup_kernel(locT, tableT)
    return jnp.transpose(outT, (2, 0, 1))
